# Initial kernel scaffold; baseline (speedup 1.0000x reference)
import functools

import jax
import jax.numpy as jnp
from jax import lax
from jax.experimental import pallas as pl
from jax.experimental.pallas import tpu as pltpu
from jax.experimental.pallas import tpu_sc as plsc

_NC = 2
_NS = 16
_NW = _NC * _NS
_CHUNK = 80


@functools.lru_cache(maxsize=None)
def _make_scatter(n_nodes, n_feat, n_edges):
    edges_per_tile = n_edges // _NW
    n_chunks = edges_per_tile // _CHUNK
    rows_per_tile = n_nodes // _NS
    zrows = 125
    assert edges_per_tile * _NW == n_edges
    assert n_chunks * _CHUNK == edges_per_tile
    assert rows_per_tile * _NS == n_nodes
    assert rows_per_tile % zrows == 0

    mesh = plsc.VectorSubcoreMesh(core_axis_name="c", subcore_axis_name="s")

    @functools.partial(
        pl.kernel,
        mesh=mesh,
        out_type=jax.ShapeDtypeStruct((_NC, n_nodes, n_feat), jnp.float32),
        scratch_types=[
            pltpu.VMEM_SHARED((n_nodes, n_feat), jnp.float32),
            pltpu.VMEM((n_chunks, _CHUNK), jnp.int32),
            pltpu.VMEM((n_chunks, _CHUNK), jnp.int32),
            pltpu.VMEM((_CHUNK, n_feat), jnp.float32),
            pltpu.VMEM((zrows, n_feat), jnp.float32),
            pltpu.SemaphoreType.DMA,
        ],
    )
    def scatter(x_hbm, row_hbm, col_hbm, out_hbm,
                acc, ridx, cidx, gbuf, zbuf, sem):
        c = lax.axis_index("c")
        s = lax.axis_index("s")
        wid = c * _NS + s

        zero = jnp.zeros((16,), jnp.float32)

        def zrow(i, _):
            def zcol(j, _):
                zbuf[i, pl.ds(j * 16, 16)] = zero
                return 0
            return lax.fori_loop(0, n_feat // 16, zcol, 0)

        lax.fori_loop(0, zrows, zrow, 0)
        row_base = s * rows_per_tile
        for k in range(rows_per_tile // zrows):
            pltpu.sync_copy(zbuf, acc.at[pl.ds(row_base + k * zrows, zrows)])

        pltpu.sync_copy(row_hbm.at[wid], ridx)
        pltpu.sync_copy(col_hbm.at[wid], cidx)
        plsc.subcore_barrier()

        def chunk(i, _):
            pltpu.async_copy(x_hbm.at[cidx.at[i]], gbuf, sem).wait()
            pltpu.sync_copy(gbuf, acc.at[ridx.at[i]], add=True)
            return 0

        lax.fori_loop(0, n_chunks, chunk, 0)
        plsc.subcore_barrier()

        for k in range(rows_per_tile // zrows):
            r0 = row_base + k * zrows
            pltpu.sync_copy(acc.at[pl.ds(r0, zrows)],
                            out_hbm.at[c, pl.ds(r0, zrows)])

    return scatter


@functools.lru_cache(maxsize=None)
def _make_combine(n_nodes, n_feat, blk):
    def body(p_ref, w_ref, b_ref, o_ref):
        agg = p_ref[0] + p_ref[1]
        o_ref[...] = jnp.maximum(
            jnp.dot(agg, w_ref[...], preferred_element_type=jnp.float32)
            + b_ref[...], 0.0)

    return pl.pallas_call(
        body,
        grid=(n_nodes // blk,),
        in_specs=[
            pl.BlockSpec((2, blk, n_feat), lambda i: (0, i, 0)),
            pl.BlockSpec((n_feat, n_feat), lambda i: (0, 0)),
            pl.BlockSpec((1, n_feat), lambda i: (0, 0)),
        ],
        out_specs=pl.BlockSpec((blk, n_feat), lambda i: (i, 0)),
        out_shape=jax.ShapeDtypeStruct((n_nodes, n_feat), jnp.float32),
    )


def kernel(x, edge_index, weight, bias):
    n_nodes, in_feat = x.shape
    n_edges = edge_index.shape[1]
    ei = edge_index.astype(jnp.int32)
    edges_per_tile = n_edges // _NW
    n_chunks = edges_per_tile // _CHUNK
    row3 = ei[0].reshape(_NW, n_chunks, _CHUNK)
    col3 = ei[1].reshape(_NW, n_chunks, _CHUNK)
    partials = _make_scatter(n_nodes, in_feat, n_edges)(x, row3, col3)
    return _make_combine(n_nodes, weight.shape[1], 2000)(
        partials, weight, bias.reshape(1, -1))


# speedup vs baseline: 7.1577x; 7.1577x over previous
import functools

import jax
import jax.numpy as jnp
from jax import lax
from jax.experimental import pallas as pl
from jax.experimental.pallas import tpu as pltpu
from jax.experimental.pallas import tpu_sc as plsc

_NC = 2
_NS = 16
_NW = _NC * _NS
_CHUNK = 80


@functools.lru_cache(maxsize=None)
def _make_scatter(n_nodes, n_feat, n_edges):
    edges_per_tile = n_edges // _NW
    n_chunks = edges_per_tile // _CHUNK
    rows_per_tile = -(-n_nodes // (_NS * _CHUNK)) * _CHUNK
    n_pad = rows_per_tile * _NS
    assert edges_per_tile * _NW == n_edges
    assert n_chunks * _CHUNK == edges_per_tile

    mesh = plsc.VectorSubcoreMesh(core_axis_name="c", subcore_axis_name="s")

    @functools.partial(
        pl.kernel,
        mesh=mesh,
        out_type=jax.ShapeDtypeStruct((_NC, n_pad, n_feat), jnp.float32),
        scratch_types=[
            pltpu.VMEM_SHARED((n_pad, n_feat), jnp.float32),
            pltpu.VMEM((n_chunks, _CHUNK), jnp.int32),
            pltpu.VMEM((n_chunks, _CHUNK), jnp.int32),
            pltpu.VMEM((_CHUNK, n_feat), jnp.float32),
            pltpu.SemaphoreType.DMA,
        ],
    )
    def scatter(x_hbm, row_hbm, col_hbm, out_hbm,
                acc, ridx, cidx, gbuf, sem):
        c = lax.axis_index("c")
        s = lax.axis_index("s")
        wid = c * _NS + s

        zero = jnp.zeros((16,), jnp.float32)

        def zrow(i, _):
            def zcol(j, _):
                gbuf[i, pl.ds(j * 16, 16)] = zero
                return 0
            return lax.fori_loop(0, n_feat // 16, zcol, 0)

        lax.fori_loop(0, _CHUNK, zrow, 0)
        row_base = s * rows_per_tile
        for k in range(rows_per_tile // _CHUNK):
            pltpu.sync_copy(gbuf, acc.at[pl.ds(row_base + k * _CHUNK, _CHUNK)])

        pltpu.sync_copy(row_hbm.at[wid], ridx)
        pltpu.sync_copy(col_hbm.at[wid], cidx)
        plsc.subcore_barrier()

        def chunk(i, _):
            pltpu.async_copy(x_hbm.at[cidx.at[i]], gbuf, sem).wait()
            pltpu.sync_copy(gbuf, acc.at[ridx.at[i]], add=True)
            return 0

        lax.fori_loop(0, n_chunks, chunk, 0)
        plsc.subcore_barrier()

        for k in range(rows_per_tile // _CHUNK):
            r0 = row_base + k * _CHUNK
            pltpu.sync_copy(acc.at[pl.ds(r0, _CHUNK)],
                            out_hbm.at[c, pl.ds(r0, _CHUNK)])

    return scatter


@functools.lru_cache(maxsize=None)
def _make_combine(n_nodes, n_feat, blk):
    def body(p_ref, w_ref, b_ref, o_ref):
        agg = p_ref[0] + p_ref[1]
        o_ref[...] = jnp.maximum(
            jnp.dot(agg, w_ref[...], preferred_element_type=jnp.float32)
            + b_ref[...], 0.0)

    return pl.pallas_call(
        body,
        grid=(n_nodes // blk,),
        in_specs=[
            pl.BlockSpec((2, blk, n_feat), lambda i: (0, i, 0)),
            pl.BlockSpec((n_feat, n_feat), lambda i: (0, 0)),
            pl.BlockSpec((1, n_feat), lambda i: (0, 0)),
        ],
        out_specs=pl.BlockSpec((blk, n_feat), lambda i: (i, 0)),
        out_shape=jax.ShapeDtypeStruct((n_nodes, n_feat), jnp.float32),
    )


def kernel(x, edge_index, weight, bias):
    n_nodes, in_feat = x.shape
    n_edges = edge_index.shape[1]
    ei = edge_index.astype(jnp.int32)
    edges_per_tile = n_edges // _NW
    n_chunks = edges_per_tile // _CHUNK
    row3 = ei[0].reshape(_NW, n_chunks, _CHUNK)
    col3 = ei[1].reshape(_NW, n_chunks, _CHUNK)
    partials = _make_scatter(n_nodes, in_feat, n_edges)(x, row3, col3)
    return _make_combine(n_nodes, weight.shape[1], 2000)(
        partials, weight, bias.reshape(1, -1))
